# baseline (speedup 1.0000x reference)
import functools

import jax
import jax.numpy as jnp
from jax.experimental import pallas as pl
from jax.experimental.pallas import tpu as pltpu

_G = 2
_CH = 128


def _fold_bn(w, gamma, beta, mean, var, eps=1e-5):
    scale = gamma / jnp.sqrt(var + eps)
    w_eff = w * scale.reshape((-1,) + (1,) * (w.ndim - 1))
    b_eff = beta - mean * scale
    return w_eff, b_eff


def _fused_kernel(x_ref, w1_ref, b1_ref, w2b_ref, b2b_ref, o_ref,
                  xp_ref, sbl_ref, sbc_ref, sbr_ref, *,
                  g, cin, c1, H, W, pad, ch):
    HW = H * W
    nck = HW // ch
    w_idx = jax.lax.broadcasted_iota(jnp.int32, (c1, ch), 1) % W
    mask_l = w_idx > 0
    mask_r = w_idx < W - 1

    for i in range(g):
        y1 = jnp.dot(w1_ref[...], x_ref[i * cin:(i + 1) * cin, :],
                     preferred_element_type=jnp.float32)
        y1 = jnp.maximum(y1 + b1_ref[...], 0.0)
        o_ref[i * 2 * c1:i * 2 * c1 + c1, :] = y1.astype(o_ref.dtype)
        xp_ref[i, :, pad:pad + HW] = y1
        xp_ref[i, :, pad - W:pad] = jnp.zeros((c1, W), jnp.float32)
        xp_ref[i, :, pad + HW:pad + HW + W] = jnp.zeros((c1, W), jnp.float32)

    def wb(t):
        return w2b_ref[:, t * ch:(t + 1) * ch]

    for i in range(g):
        for c in range(nck):
            base = pad + c * ch
            up = xp_ref[i, :, base - W:base - W + ch]
            md = xp_ref[i, :, base:base + ch]
            dn = xp_ref[i, :, base + W:base + W + ch]
            sbl_ref[i, :, base:base + ch] = (
                wb(0) * up + wb(3) * md + wb(6) * dn)
            sbc_ref[i, :, base:base + ch] = (
                wb(1) * up + wb(4) * md + wb(7) * dn)
            sbr_ref[i, :, base:base + ch] = (
                wb(2) * up + wb(5) * md + wb(8) * dn)

    for i in range(g):
        for c in range(nck):
            base = pad + c * ch
            bl = sbl_ref[i, :, base - 1:base - 1 + ch]
            bc = sbc_ref[i, :, base:base + ch]
            br = sbr_ref[i, :, base + 1:base + 1 + ch]
            y2 = (bc
                  + jnp.where(mask_l, bl, 0.0)
                  + jnp.where(mask_r, br, 0.0))
            y2 = jnp.maximum(y2 + b2b_ref[:, 0:ch], 0.0)
            o_ref[i * 2 * c1 + c1:(i + 1) * 2 * c1, c * ch:c * ch + ch] = (
                y2.astype(o_ref.dtype))


def kernel(x, w_primary, bn1_gamma, bn1_beta, bn1_mean, bn1_var,
           w_dw, bn2_gamma, bn2_beta, bn2_mean, bn2_var):
    B, cin, H, W = x.shape
    HW = H * W
    c1 = w_primary.shape[0]
    G = _G
    while B % G:
        G //= 2
    ch = _CH if HW % _CH == 0 else HW
    pad = 128

    w1, b1 = _fold_bn(w_primary.reshape(c1, cin),
                      bn1_gamma, bn1_beta, bn1_mean, bn1_var)
    w2, b2 = _fold_bn(w_dw.reshape(c1, 9),
                      bn2_gamma, bn2_beta, bn2_mean, bn2_var)
    w1 = w1.astype(jnp.float32)
    b1 = b1.reshape(c1, 1).astype(jnp.float32)
    w2b = jnp.repeat(w2.astype(jnp.float32), ch, axis=1)
    b2b = jnp.broadcast_to(b2.reshape(c1, 1).astype(jnp.float32), (c1, ch))

    lin = HW + 2 * pad
    x3 = x.reshape(B // G, G * cin, HW)
    out = pl.pallas_call(
        functools.partial(_fused_kernel, g=G, cin=cin, c1=c1, H=H, W=W,
                          pad=pad, ch=ch),
        out_shape=jax.ShapeDtypeStruct((B // G, G * 2 * c1, HW), x.dtype),
        grid=(B // G,),
        in_specs=[
            pl.BlockSpec((None, G * cin, HW), lambda b: (b, 0, 0)),
            pl.BlockSpec((c1, cin), lambda b: (0, 0)),
            pl.BlockSpec((c1, 1), lambda b: (0, 0)),
            pl.BlockSpec((c1, 9 * ch), lambda b: (0, 0)),
            pl.BlockSpec((c1, ch), lambda b: (0, 0)),
        ],
        out_specs=pl.BlockSpec((None, G * 2 * c1, HW), lambda b: (b, 0, 0)),
        scratch_shapes=[pltpu.VMEM((G, c1, lin), jnp.float32),
                        pltpu.VMEM((G, c1, lin), jnp.float32),
                        pltpu.VMEM((G, c1, lin), jnp.float32),
                        pltpu.VMEM((G, c1, lin), jnp.float32)],
        compiler_params=pltpu.CompilerParams(
            dimension_semantics=("parallel",)),
        cost_estimate=pl.CostEstimate(
            flops=int(2 * B * HW * cin * c1 + 2 * B * c1 * HW * 9),
            transcendentals=0,
            bytes_accessed=int(4 * (B * cin * HW + B * 2 * c1 * HW))),
    )(x3, w1, b1, w2b, b2b)
    return out.reshape(B, 2 * c1, H, W)


# speedup vs baseline: 1.9988x; 1.9988x over previous
import functools

import jax
import jax.numpy as jnp
from jax.experimental import pallas as pl
from jax.experimental.pallas import tpu as pltpu

_G = 1
_CH = 128


def _fold_bn(w, gamma, beta, mean, var, eps=1e-5):
    scale = gamma / jnp.sqrt(var + eps)
    w_eff = w * scale.reshape((-1,) + (1,) * (w.ndim - 1))
    b_eff = beta - mean * scale
    return w_eff, b_eff


def _fused_kernel(x_ref, w1_ref, b1_ref, w2b_ref, b2b_ref, o_ref,
                  xp_ref, sbl_ref, sbc_ref, sbr_ref, *,
                  g, cin, c1, H, W, pad, ch):
    HW = H * W
    nck = HW // ch
    w_idx = jax.lax.broadcasted_iota(jnp.int32, (c1, ch), 1) % W
    mask_l = w_idx > 0
    mask_r = w_idx < W - 1

    for i in range(g):
        y1 = jnp.dot(w1_ref[...], x_ref[i * cin:(i + 1) * cin, :],
                     preferred_element_type=jnp.float32)
        y1 = jnp.maximum(y1 + b1_ref[...], 0.0)
        o_ref[i * 2 * c1:i * 2 * c1 + c1, :] = y1.astype(o_ref.dtype)
        xp_ref[i, :, pad:pad + HW] = y1
        xp_ref[i, :, pad - W:pad] = jnp.zeros((c1, W), jnp.float32)
        xp_ref[i, :, pad + HW:pad + HW + W] = jnp.zeros((c1, W), jnp.float32)

    def wb(t):
        return w2b_ref[:, t * ch:(t + 1) * ch]

    for i in range(g):
        for c in range(nck):
            base = pad + c * ch
            up = xp_ref[i, :, base - W:base - W + ch]
            md = xp_ref[i, :, base:base + ch]
            dn = xp_ref[i, :, base + W:base + W + ch]
            sbl_ref[i, :, base:base + ch] = (
                wb(0) * up + wb(3) * md + wb(6) * dn)
            sbc_ref[i, :, base:base + ch] = (
                wb(1) * up + wb(4) * md + wb(7) * dn)
            sbr_ref[i, :, base:base + ch] = (
                wb(2) * up + wb(5) * md + wb(8) * dn)

    for i in range(g):
        for c in range(nck):
            base = pad + c * ch
            bl = sbl_ref[i, :, base - 1:base - 1 + ch]
            bc = sbc_ref[i, :, base:base + ch]
            br = sbr_ref[i, :, base + 1:base + 1 + ch]
            y2 = (bc
                  + jnp.where(mask_l, bl, 0.0)
                  + jnp.where(mask_r, br, 0.0))
            y2 = jnp.maximum(y2 + b2b_ref[:, 0:ch], 0.0)
            o_ref[i * 2 * c1 + c1:(i + 1) * 2 * c1, c * ch:c * ch + ch] = (
                y2.astype(o_ref.dtype))


def kernel(x, w_primary, bn1_gamma, bn1_beta, bn1_mean, bn1_var,
           w_dw, bn2_gamma, bn2_beta, bn2_mean, bn2_var):
    B, cin, H, W = x.shape
    HW = H * W
    c1 = w_primary.shape[0]
    G = _G
    while B % G:
        G //= 2
    ch = _CH if HW % _CH == 0 else HW
    pad = 128

    w1, b1 = _fold_bn(w_primary.reshape(c1, cin),
                      bn1_gamma, bn1_beta, bn1_mean, bn1_var)
    w2, b2 = _fold_bn(w_dw.reshape(c1, 9),
                      bn2_gamma, bn2_beta, bn2_mean, bn2_var)
    w1 = w1.astype(jnp.float32)
    b1 = b1.reshape(c1, 1).astype(jnp.float32)
    w2b = jnp.repeat(w2.astype(jnp.float32), ch, axis=1)
    b2b = jnp.broadcast_to(b2.reshape(c1, 1).astype(jnp.float32), (c1, ch))

    lin = HW + 2 * pad
    x3 = x.reshape(B // G, G * cin, HW)
    out = pl.pallas_call(
        functools.partial(_fused_kernel, g=G, cin=cin, c1=c1, H=H, W=W,
                          pad=pad, ch=ch),
        out_shape=jax.ShapeDtypeStruct((B // G, G * 2 * c1, HW), x.dtype),
        grid=(B // G,),
        in_specs=[
            pl.BlockSpec((None, G * cin, HW), lambda b: (b, 0, 0)),
            pl.BlockSpec((c1, cin), lambda b: (0, 0)),
            pl.BlockSpec((c1, 1), lambda b: (0, 0)),
            pl.BlockSpec((c1, 9 * ch), lambda b: (0, 0)),
            pl.BlockSpec((c1, ch), lambda b: (0, 0)),
        ],
        out_specs=pl.BlockSpec((None, G * 2 * c1, HW), lambda b: (b, 0, 0)),
        scratch_shapes=[pltpu.VMEM((G, c1, lin), jnp.float32),
                        pltpu.VMEM((G, c1, lin), jnp.float32),
                        pltpu.VMEM((G, c1, lin), jnp.float32),
                        pltpu.VMEM((G, c1, lin), jnp.float32)],
        compiler_params=pltpu.CompilerParams(
            dimension_semantics=("parallel",)),
        cost_estimate=pl.CostEstimate(
            flops=int(2 * B * HW * cin * c1 + 2 * B * c1 * HW * 9),
            transcendentals=0,
            bytes_accessed=int(4 * (B * cin * HW + B * 2 * c1 * HW))),
    )(x3, w1, b1, w2b, b2b)
    return out.reshape(B, 2 * c1, H, W)
